# baseline (speedup 1.0000x reference)
import functools

import jax
import jax.numpy as jnp
from jax import lax
from jax.experimental import pallas as pl
from jax.experimental.pallas import tpu as pltpu
from jax.experimental.pallas import tpu_sc as plsc

_B, _F, _T = 128, 128, 1024
_NW = 32
_SPW = _B // _NW
_CH = 16
_NCH = _F // _CH
_NBUF = 6
_LANES = 16
_MAXW = 7


def _sc_body(lo_hbm, hi_hbm, x_hbm, o_hbm, lo_v, hi_v, zrow,
             b0, b1, b2, b3, b4, b5, gsems, ssems, zsem):
    cid = lax.axis_index("c")
    sid = lax.axis_index("s")
    wid = cid * 16 + sid
    bufs = (b0, b1, b2, b3, b4, b5)

    pltpu.sync_copy(lo_hbm, lo_v.at[pl.ds(0, _B)])
    pltpu.sync_copy(hi_hbm, hi_v.at[pl.ds(0, _B)])
    lo_vec = lo_v[pl.ds(wid * _SPW, _LANES)]
    hi_vec = hi_v[pl.ds(wid * _SPW, _LANES)]

    zvec = jnp.zeros((_LANES,), jnp.float32)
    for seg in range(_T // _LANES):
        zrow[0, pl.ds(seg * _LANES, _LANES)] = zvec

    def gather(k):
        b = wid * _SPW + k // _NCH
        c = k % _NCH
        slot = k % _NBUF
        return pltpu.make_async_copy(
            x_hbm.at[b, pl.ds(c * _CH, _CH)], bufs[slot], gsems.at[slot]
        )

    def scatter(k):
        b = wid * _SPW + k // _NCH
        c = k % _NCH
        slot = k % _NBUF
        return pltpu.make_async_copy(
            bufs[slot], o_hbm.at[b, pl.ds(c * _CH, _CH)], ssems.at[slot]
        )

    def zero_band(i):
        b = wid * _SPW + i
        lo = lo_vec[i]
        last = hi_vec[i] - 1
        for j in range(_MAXW):
            row = jnp.minimum(lo + j, last)
            pltpu.make_async_copy(
                zrow, o_hbm.at[b, pl.ds(row, 1)], zsem
            ).start()

    nk = _SPW * _NCH
    for k in range(min(3, nk)):
        gather(k).start()

    for k in range(nk):
        gather(k).wait()
        scatter(k).start()
        if k >= 2:
            scatter(k - 2).wait()
        if k >= 2 and (k - 2) % _NCH == _NCH - 1:
            zero_band((k - 2) // _NCH)
        if k + 3 < nk:
            gather(k + 3).start()
    scatter(nk - 2).wait()
    scatter(nk - 1).wait()
    zero_band(_SPW - 1)

    dummy = pltpu.make_async_copy(x_hbm.at[0, pl.ds(0, 1)], zrow, zsem)
    for _ in range(_SPW * _MAXW):
        dummy.wait()


def kernel(x):
    mask_ratio = 16
    xs = jnp.squeeze(x, axis=1)
    B, F, T = xs.shape
    max_mask = F // mask_ratio
    k = jax.random.key(42)
    k1, k2 = jax.random.split(k)
    if max_mask == 1:
        f_widths = jnp.ones((B,), dtype=jnp.int32)
    else:
        f_widths = jax.random.randint(k1, (B,), 1, max_mask).astype(jnp.int32)
    u = jax.random.uniform(k2, (B,))
    f_low = jnp.floor(u * (F - f_widths).astype(jnp.float32)).astype(jnp.int32)
    f_hi = f_low + f_widths

    run = functools.partial(
        pl.kernel,
        out_type=jax.ShapeDtypeStruct((B, F, T), jnp.float32),
        mesh=plsc.VectorSubcoreMesh(core_axis_name="c", subcore_axis_name="s"),
        compiler_params=pltpu.CompilerParams(use_tc_tiling_on_sc=True),
        scratch_types=[
            pltpu.VMEM((_B + _LANES,), jnp.int32),
            pltpu.VMEM((_B + _LANES,), jnp.int32),
            pltpu.VMEM((1, _T), jnp.float32),
            pltpu.VMEM((_CH, _T), jnp.float32),
            pltpu.VMEM((_CH, _T), jnp.float32),
            pltpu.VMEM((_CH, _T), jnp.float32),
            pltpu.VMEM((_CH, _T), jnp.float32),
            pltpu.VMEM((_CH, _T), jnp.float32),
            pltpu.VMEM((_CH, _T), jnp.float32),
            pltpu.SemaphoreType.DMA((_NBUF,)),
            pltpu.SemaphoreType.DMA((_NBUF,)),
            pltpu.SemaphoreType.DMA,
        ],
    )(_sc_body)
    out = run(f_low, f_hi, xs)
    return out[:, None, :, :]


# speedup vs baseline: 1.2166x; 1.2166x over previous
import functools

import jax
import jax.numpy as jnp
from jax import lax
from jax.experimental import pallas as pl
from jax.experimental.pallas import tpu as pltpu
from jax.experimental.pallas import tpu_sc as plsc

_B, _F, _T = 128, 128, 1024
_NW = 32
_SPW = _B // _NW
_CH = 16
_NCH = _F // _CH
_NBUF = 7
_LANES = 16


def _sc_body(lo_hbm, hi_hbm, x_hbm, o_hbm, lo_v, hi_v,
             b0, b1, b2, b3, b4, b5, b6, gsems, ssems):
    cid = lax.axis_index("c")
    sid = lax.axis_index("s")
    wid = cid * 16 + sid
    bufs = (b0, b1, b2, b3, b4, b5, b6)

    pltpu.sync_copy(lo_hbm, lo_v.at[pl.ds(0, _B)])
    pltpu.sync_copy(hi_hbm, hi_v.at[pl.ds(0, _B)])
    lo_vec = lo_v[pl.ds(wid * _SPW, _LANES)]
    hi_vec = hi_v[pl.ds(wid * _SPW, _LANES)]

    zvec = jnp.zeros((_LANES,), jnp.float32)

    def gather(k):
        b = wid * _SPW + k // _NCH
        c = k % _NCH
        slot = k % _NBUF
        return pltpu.make_async_copy(
            x_hbm.at[b, pl.ds(c * _CH, _CH)], bufs[slot], gsems.at[slot]
        )

    def scatter(k):
        b = wid * _SPW + k // _NCH
        c = k % _NCH
        slot = k % _NBUF
        return pltpu.make_async_copy(
            bufs[slot], o_hbm.at[b, pl.ds(c * _CH, _CH)], ssems.at[slot]
        )

    nk = _SPW * _NCH
    for k in range(min(3, nk)):
        gather(k).start()

    lo = hi = None
    for k in range(nk):
        c = k % _NCH
        if c == 0:
            lo = lo_vec[k // _NCH]
            hi = hi_vec[k // _NCH]
        gather(k).wait()
        c0 = c * _CH
        s = jnp.clip(lo, c0, c0 + _CH) - c0
        e = jnp.clip(hi, c0, c0 + _CH) - c0
        buf = bufs[k % _NBUF]

        def zero_row(r, _, buf=buf):
            for seg in range(_T // _LANES):
                buf[r, pl.ds(seg * _LANES, _LANES)] = zvec
            return 0

        lax.fori_loop(s, e, zero_row, 0)
        scatter(k).start()
        if k >= 3:
            scatter(k - 3).wait()
        if k + 3 < nk:
            gather(k + 3).start()
    for k in range(max(nk - 3, 0), nk):
        scatter(k).wait()


def kernel(x):
    mask_ratio = 16
    xs = jnp.squeeze(x, axis=1)
    B, F, T = xs.shape
    max_mask = F // mask_ratio
    with jax.ensure_compile_time_eval():
        k = jax.random.key(42)
        k1, k2 = jax.random.split(k)
        if max_mask == 1:
            f_widths = jnp.ones((B,), dtype=jnp.int32)
        else:
            f_widths = jax.random.randint(k1, (B,), 1, max_mask).astype(jnp.int32)
        u = jax.random.uniform(k2, (B,))
        f_low = jnp.floor(u * (F - f_widths).astype(jnp.float32)).astype(jnp.int32)
        f_hi = f_low + f_widths

    run = functools.partial(
        pl.kernel,
        out_type=jax.ShapeDtypeStruct((B, F, T), jnp.float32),
        mesh=plsc.VectorSubcoreMesh(core_axis_name="c", subcore_axis_name="s"),
        compiler_params=pltpu.CompilerParams(use_tc_tiling_on_sc=True),
        scratch_types=[
            pltpu.VMEM((_B + _LANES,), jnp.int32),
            pltpu.VMEM((_B + _LANES,), jnp.int32),
            pltpu.VMEM((_CH, _T), jnp.float32),
            pltpu.VMEM((_CH, _T), jnp.float32),
            pltpu.VMEM((_CH, _T), jnp.float32),
            pltpu.VMEM((_CH, _T), jnp.float32),
            pltpu.VMEM((_CH, _T), jnp.float32),
            pltpu.VMEM((_CH, _T), jnp.float32),
            pltpu.VMEM((_CH, _T), jnp.float32),
            pltpu.SemaphoreType.DMA((_NBUF,)),
            pltpu.SemaphoreType.DMA((_NBUF,)),
        ],
    )(_sc_body)
    out = run(f_low, f_hi, xs)
    return out[:, None, :, :]


# speedup vs baseline: 1.2249x; 1.0068x over previous
import functools

import jax
import jax.numpy as jnp
from jax import lax
from jax.experimental import pallas as pl
from jax.experimental.pallas import tpu as pltpu
from jax.experimental.pallas import tpu_sc as plsc

_B, _F, _T = 128, 128, 1024
_NW = 32
_SPW = _B // _NW
_CH = 16
_NCH = _F // _CH
_NBUF = 7
_LANES = 16


def _sc_body(band_hbm, x_hbm, o_hbm, band_v,
             b0, b1, b2, b3, b4, b5, b6, gsems, ssems):
    cid = lax.axis_index("c")
    sid = lax.axis_index("s")
    wid = cid * 16 + sid
    bufs = (b0, b1, b2, b3, b4, b5, b6)

    pltpu.sync_copy(band_hbm, band_v.at[pl.ds(0, _B)])
    band_vec = band_v[pl.ds(wid * _SPW, _LANES)]
    lo_vec = jnp.bitwise_and(band_vec, 0xFFFF)
    hi_vec = jnp.right_shift(band_vec, 16)

    zvec = jnp.zeros((_LANES,), jnp.float32)

    def gather(k):
        b = wid * _SPW + k // _NCH
        c = k % _NCH
        slot = k % _NBUF
        return pltpu.make_async_copy(
            x_hbm.at[b, pl.ds(c * _CH, _CH)], bufs[slot], gsems.at[slot]
        )

    def scatter(k):
        b = wid * _SPW + k // _NCH
        c = k % _NCH
        slot = k % _NBUF
        return pltpu.make_async_copy(
            bufs[slot], o_hbm.at[b, pl.ds(c * _CH, _CH)], ssems.at[slot]
        )

    nk = _SPW * _NCH
    for k in range(min(2, nk)):
        gather(k).start()

    lo = hi = None
    for k in range(nk):
        c = k % _NCH
        if c == 0:
            lo = lo_vec[k // _NCH]
            hi = hi_vec[k // _NCH]
        gather(k).wait()
        c0 = c * _CH
        s = jnp.clip(lo, c0, c0 + _CH) - c0
        e = jnp.clip(hi, c0, c0 + _CH) - c0
        buf = bufs[k % _NBUF]

        def zero_row(r, _, buf=buf):
            for seg in range(_T // _LANES):
                buf[r, pl.ds(seg * _LANES, _LANES)] = zvec
            return 0

        lax.fori_loop(s, e, zero_row, 0)
        scatter(k).start()
        if k >= 4:
            scatter(k - 4).wait()
        if k + 2 < nk:
            gather(k + 2).start()
    for k in range(max(nk - 4, 0), nk):
        scatter(k).wait()


def kernel(x):
    mask_ratio = 16
    xs = jnp.squeeze(x, axis=1)
    B, F, T = xs.shape
    max_mask = F // mask_ratio
    with jax.ensure_compile_time_eval():
        k = jax.random.key(42)
        k1, k2 = jax.random.split(k)
        if max_mask == 1:
            f_widths = jnp.ones((B,), dtype=jnp.int32)
        else:
            f_widths = jax.random.randint(k1, (B,), 1, max_mask).astype(jnp.int32)
        u = jax.random.uniform(k2, (B,))
        f_low = jnp.floor(u * (F - f_widths).astype(jnp.float32)).astype(jnp.int32)
        f_hi = f_low + f_widths
        band = f_low | (f_hi << 16)

    run = functools.partial(
        pl.kernel,
        out_type=jax.ShapeDtypeStruct((B, F, T), jnp.float32),
        mesh=plsc.VectorSubcoreMesh(core_axis_name="c", subcore_axis_name="s"),
        compiler_params=pltpu.CompilerParams(use_tc_tiling_on_sc=True),
        scratch_types=[
            pltpu.VMEM((_B + _LANES,), jnp.int32),
            pltpu.VMEM((_CH, _T), jnp.float32),
            pltpu.VMEM((_CH, _T), jnp.float32),
            pltpu.VMEM((_CH, _T), jnp.float32),
            pltpu.VMEM((_CH, _T), jnp.float32),
            pltpu.VMEM((_CH, _T), jnp.float32),
            pltpu.VMEM((_CH, _T), jnp.float32),
            pltpu.VMEM((_CH, _T), jnp.float32),
            pltpu.SemaphoreType.DMA((_NBUF,)),
            pltpu.SemaphoreType.DMA((_NBUF,)),
        ],
    )(_sc_body)
    out = run(band, xs)
    return out[:, None, :, :]


# speedup vs baseline: 2.0607x; 1.6823x over previous
import functools

import jax
import jax.numpy as jnp
from jax import lax
from jax.experimental import pallas as pl
from jax.experimental.pallas import tpu as pltpu

_BB = 16


def _mask_kernel(lo_ref, hi_ref, x_ref, o_ref, *, bb, F, T):
    i = pl.program_id(0)
    rows = lax.broadcasted_iota(jnp.int32, (F, T), 0)
    for j in range(bb):
        lo = lo_ref[i * bb + j]
        hi = hi_ref[i * bb + j]
        band = (rows >= lo) & (rows < hi)
        o_ref[j] = jnp.where(band, jnp.float32(0.0), x_ref[j])


def kernel(x):
    mask_ratio = 16
    xs = jnp.squeeze(x, axis=1)
    B, F, T = xs.shape
    max_mask = F // mask_ratio
    with jax.ensure_compile_time_eval():
        k = jax.random.key(42)
        k1, k2 = jax.random.split(k)
        if max_mask == 1:
            f_widths = jnp.ones((B,), dtype=jnp.int32)
        else:
            f_widths = jax.random.randint(k1, (B,), 1, max_mask).astype(jnp.int32)
        u = jax.random.uniform(k2, (B,))
        f_low = jnp.floor(u * (F - f_widths).astype(jnp.float32)).astype(jnp.int32)
        f_hi = f_low + f_widths

    bb = _BB
    grid = (B // bb,)
    out = pl.pallas_call(
        functools.partial(_mask_kernel, bb=bb, F=F, T=T),
        grid_spec=pltpu.PrefetchScalarGridSpec(
            num_scalar_prefetch=2,
            grid=grid,
            in_specs=[
                pl.BlockSpec((bb, F, T), lambda i, lo, hi: (i, 0, 0)),
            ],
            out_specs=pl.BlockSpec((bb, F, T), lambda i, lo, hi: (i, 0, 0)),
        ),
        out_shape=jax.ShapeDtypeStruct((B, F, T), jnp.float32),
    )(f_low, f_hi, xs)
    return out[:, None, :, :]
